# baseline (speedup 1.0000x reference)
import functools

import jax
import jax.numpy as jnp
from jax import lax
from jax.experimental import pallas as pl
from jax.experimental.pallas import tpu as pltpu
from jax.experimental.pallas import tpu_sc as plsc

N = 10000
K = 32
D = 128
D_OUT = 128

NC = 2
NS = 16
NW = NC * NS
N_PAD = 10240
NODES_PW = N_PAD // NW
NODES_SC = NS * NODES_PW
CHUNK = 4
ROWS_PC = CHUNK * K
NCH = NODES_PW // CHUNK
NB = 2
OUT_T = 64
NV = D // 16


def _sc_body(x_hbm, adj_hbm, dst_hbm, out_hbm,
             idx_v, dst_v, b0, b1, tmp_v, acc_s,
             s0, s1):
    cc = lax.axis_index("c")
    ss = lax.axis_index("s")
    wid = cc * NS + ss
    bufs = (b0, b1)
    sems = (s0, s1)

    pltpu.sync_copy(adj_hbm.at[pl.ds(wid * NCH, NCH)], idx_v)
    pltpu.sync_copy(dst_hbm.at[pl.ds(wid * NCH, NCH)], dst_v)

    def zrow(r, _):
        for d in range(NV):
            tmp_v[r, pl.ds(d * 16, 16)] = jnp.zeros((16,), jnp.float32)
        return _

    lax.fori_loop(0, OUT_T, zrow, 0)
    for j in range(NODES_PW // OUT_T):
        pltpu.sync_copy(tmp_v, acc_s.at[pl.ds(ss * NODES_PW + j * OUT_T, OUT_T)])

    for b in range(NB):
        pltpu.async_copy(x_hbm.at[idx_v.at[b]], bufs[b], sems[b])

    def outer(g, carry):
        for b in range(NB):
            ch = g * NB + b
            buf, sem = bufs[b], sems[b]
            pltpu.make_async_copy(x_hbm.at[idx_v.at[ch]], buf, sem).wait()
            pltpu.sync_copy(buf, acc_s.at[dst_v.at[ch]], add=True)

            @pl.when(ch + NB < NCH)
            def _():
                pltpu.async_copy(x_hbm.at[idx_v.at[ch + NB]], buf, sem)

        return carry

    lax.fori_loop(0, NCH // NB, outer, 0)

    scale = jnp.float32(1.0 / K)

    def drow(r, _):
        for d in range(NV):
            tmp_v[r, pl.ds(d * 16, 16)] = tmp_v[r, pl.ds(d * 16, 16)] * scale
        return _

    for j in range(NODES_PW // OUT_T):
        pltpu.sync_copy(acc_s.at[pl.ds(ss * NODES_PW + j * OUT_T, OUT_T)], tmp_v)
        lax.fori_loop(0, OUT_T, drow, 0)
        pltpu.sync_copy(
            tmp_v, out_hbm.at[pl.ds(wid * NODES_PW + j * OUT_T, OUT_T)]
        )


def _sc_gather_mean(x, adj_rows, dst_rows):
    mesh = plsc.VectorSubcoreMesh(core_axis_name="c", subcore_axis_name="s")
    f = functools.partial(
        pl.kernel,
        mesh=mesh,
        out_type=jax.ShapeDtypeStruct((N_PAD, D), jnp.float32),
        scratch_types=[
            pltpu.VMEM((NCH, ROWS_PC), jnp.int32),
            pltpu.VMEM((NCH, ROWS_PC), jnp.int32),
            pltpu.VMEM((ROWS_PC, D), jnp.float32),
            pltpu.VMEM((ROWS_PC, D), jnp.float32),
            pltpu.VMEM((OUT_T, D), jnp.float32),
            pltpu.VMEM_SHARED((NODES_SC, D), jnp.float32),
            pltpu.SemaphoreType.DMA,
            pltpu.SemaphoreType.DMA,
        ],
    )(_sc_body)
    return f(x, adj_rows, dst_rows)


BM = 1000


def _linear_body(x_ref, agg_ref, w_ref, b_ref, o_ref):
    wt = w_ref[0:D, :]
    wb = w_ref[D : 2 * D, :]
    o_ref[...] = (
        jnp.dot(x_ref[...], wt, preferred_element_type=jnp.float32)
        + jnp.dot(agg_ref[...], wb, preferred_element_type=jnp.float32)
        + b_ref[...]
    )


def _tc_linear(x, agg, W, b):
    return pl.pallas_call(
        _linear_body,
        grid=(N // BM,),
        in_specs=[
            pl.BlockSpec((BM, D), lambda i: (i, 0)),
            pl.BlockSpec((BM, D), lambda i: (i, 0)),
            pl.BlockSpec((2 * D, D_OUT), lambda i: (0, 0)),
            pl.BlockSpec((1, D_OUT), lambda i: (0, 0)),
        ],
        out_specs=pl.BlockSpec((BM, D_OUT), lambda i: (i, 0)),
        out_shape=jax.ShapeDtypeStruct((N, D_OUT), jnp.float32),
    )(x, agg, W, b.reshape(1, D_OUT))


def kernel(x, adj, W, b):
    adj_rows = jnp.pad(adj, ((0, N_PAD - N), (0, 0))).reshape(
        N_PAD // CHUNK, ROWS_PC
    )
    g = jnp.arange(N_PAD // CHUNK, dtype=jnp.int32)
    s_loc = (g // NCH) % NS
    base = s_loc * NODES_PW + (g % NCH) * CHUNK
    r = jnp.arange(ROWS_PC, dtype=jnp.int32) // K
    dst_rows = base[:, None] + r[None, :]
    agg = _sc_gather_mean(x, adj_rows, dst_rows)[:N]
    return _tc_linear(x, agg, W, b)


# speedup vs baseline: 1.0211x; 1.0211x over previous
import functools

import jax
import jax.numpy as jnp
from jax import lax
from jax.experimental import pallas as pl
from jax.experimental.pallas import tpu as pltpu
from jax.experimental.pallas import tpu_sc as plsc

N = 10000
K = 32
D = 128
D_OUT = 128

NC = 2
NS = 16
NW = NC * NS
N_PAD = 10240
NODES_PW = N_PAD // NW
CHUNK = 4
ROWS_PC = CHUNK * K
NCH = NODES_PW // CHUNK
NB = 6
NV = D // 16


def _sc_body(x_hbm, adj_hbm, out_hbm,
             idx_v, b0, b1, b2, b3, b4, b5, acc_v,
             s0, s1, s2, s3, s4, s5):
    cc = lax.axis_index("c")
    ss = lax.axis_index("s")
    wid = cc * NS + ss
    bufs = (b0, b1, b2, b3, b4, b5)
    sems = (s0, s1, s2, s3, s4, s5)

    pltpu.sync_copy(adj_hbm.at[pl.ds(wid * NCH, NCH)], idx_v)
    for b in range(NB):
        pltpu.async_copy(x_hbm.at[idx_v.at[b]], bufs[b], sems[b])

    def reduce_chunk(buf, ch):
        for nloc in range(CHUNK):
            base = nloc * K
            init = tuple(buf[base, pl.ds(d * 16, 16)] for d in range(NV))

            def kbody(kk, accs):
                return tuple(
                    accs[d] + buf[base + kk, pl.ds(d * 16, 16)]
                    for d in range(NV)
                )

            accs = lax.fori_loop(1, K, kbody, init)
            for d in range(NV):
                acc_v[nloc, pl.ds(d * 16, 16)] = accs[d] * (1.0 / K)
        pltpu.sync_copy(
            acc_v, out_hbm.at[pl.ds(wid * NODES_PW + ch * CHUNK, CHUNK)]
        )

    def outer(g, carry):
        for b in range(NB):
            ch = g * NB + b
            buf, sem = bufs[b], sems[b]
            pltpu.make_async_copy(x_hbm.at[idx_v.at[ch]], buf, sem).wait()
            reduce_chunk(buf, ch)

            @pl.when(ch + NB < NCH)
            def _():
                pltpu.async_copy(x_hbm.at[idx_v.at[ch + NB]], buf, sem)

        return carry

    lax.fori_loop(0, NCH // NB, outer, 0)
    for ch in range(NCH - NCH % NB, NCH):
        b = ch % NB
        pltpu.make_async_copy(x_hbm.at[idx_v.at[ch]], bufs[b], sems[b]).wait()
        reduce_chunk(bufs[b], ch)


def _sc_gather_mean(x, adj_rows):
    mesh = plsc.VectorSubcoreMesh(core_axis_name="c", subcore_axis_name="s")
    f = functools.partial(
        pl.kernel,
        mesh=mesh,
        out_type=jax.ShapeDtypeStruct((N_PAD, D), jnp.float32),
        scratch_types=[
            pltpu.VMEM((NCH, ROWS_PC), jnp.int32),
            pltpu.VMEM((ROWS_PC, D), jnp.float32),
            pltpu.VMEM((ROWS_PC, D), jnp.float32),
            pltpu.VMEM((ROWS_PC, D), jnp.float32),
            pltpu.VMEM((ROWS_PC, D), jnp.float32),
            pltpu.VMEM((ROWS_PC, D), jnp.float32),
            pltpu.VMEM((ROWS_PC, D), jnp.float32),
            pltpu.VMEM((CHUNK, D), jnp.float32),
            pltpu.SemaphoreType.DMA,
            pltpu.SemaphoreType.DMA,
            pltpu.SemaphoreType.DMA,
            pltpu.SemaphoreType.DMA,
            pltpu.SemaphoreType.DMA,
            pltpu.SemaphoreType.DMA,
        ],
    )(_sc_body)
    return f(x, adj_rows)


BM = 1000


def _linear_body(x_ref, agg_ref, w_ref, b_ref, o_ref):
    wt = w_ref[0:D, :]
    wb = w_ref[D : 2 * D, :]
    o_ref[...] = (
        jnp.dot(x_ref[...], wt, preferred_element_type=jnp.float32)
        + jnp.dot(agg_ref[...], wb, preferred_element_type=jnp.float32)
        + b_ref[...]
    )


def _tc_linear(x, agg, W, b):
    return pl.pallas_call(
        _linear_body,
        grid=(N // BM,),
        in_specs=[
            pl.BlockSpec((BM, D), lambda i: (i, 0)),
            pl.BlockSpec((BM, D), lambda i: (i, 0)),
            pl.BlockSpec((2 * D, D_OUT), lambda i: (0, 0)),
            pl.BlockSpec((1, D_OUT), lambda i: (0, 0)),
        ],
        out_specs=pl.BlockSpec((BM, D_OUT), lambda i: (i, 0)),
        out_shape=jax.ShapeDtypeStruct((N, D_OUT), jnp.float32),
    )(x, agg, W, b.reshape(1, D_OUT))


def kernel(x, adj, W, b):
    adj_rows = jnp.pad(adj, ((0, N_PAD - N), (0, 0))).reshape(
        N_PAD // CHUNK, ROWS_PC
    )
    agg = _sc_gather_mean(x, adj_rows)[:N]
    return _tc_linear(x, agg, W, b)
